# baseline (speedup 1.0000x reference)
import jax
import jax.numpy as jnp
from jax.experimental import pallas as pl
from jax.experimental.pallas import tpu as pltpu

BN_EPS = 1e-5
TN_EPS = 1e-4

_MM_DTYPE = jnp.float8_e4m3fn

_BM = 512
_POP_ROWS = 64
_T = 256
_HALF = 128


def _sign_pm1(x, dtype):
    return jnp.where(x >= 0, 1.0, -1.0).astype(dtype)


def _binarize_weights_body(w1_ref, w2_ref, w3_ref, w4_ref,
                           o1_ref, o2_ref, o3_ref, o4_ref):
    o1_ref[...] = _sign_pm1(w1_ref[...], _MM_DTYPE)
    o2_ref[...] = _sign_pm1(w2_ref[...], _MM_DTYPE)
    o3_ref[...] = _sign_pm1(w3_ref[...], _MM_DTYPE)
    o4_ref[...] = _sign_pm1(w4_ref[...], _MM_DTYPE)


def _mlp_body(tn_ref, x_ref, w1_ref, w2_ref, w3_ref, w4_ref, bn_ref, o_ref,
              *h_refs):
    k_tiles = x_ref.shape[1] // _T
    chunks = _BM // _POP_ROWS
    bufs = (h_refs[:4], h_refs[4:])

    def pop_unit(j, i, h_out):
        addr = (j % 2) * _HALF
        for mxu in range(2):
            n = 2 * j + mxu
            cols = n * _T
            g = bn_ref[4 * i + 0, cols:cols + _T]
            b = bn_ref[4 * i + 1, cols:cols + _T]
            m = bn_ref[4 * i + 2, cols:cols + _T]
            v = bn_ref[4 * i + 3, cols:cols + _T]
            scale = g * jax.lax.rsqrt(v + BN_EPS)
            for c in range(chunks):
                y = pltpu.matmul_pop(
                    acc_addr=addr + c * (_POP_ROWS // 4),
                    shape=(_POP_ROWS, _T), dtype=jnp.float32, mxu_index=mxu)
                t = (y - m) * scale + b
                rows = c * _POP_ROWS
                h_out[n][rows:rows + _POP_ROWS, :] = _sign_pm1(t, _MM_DTYPE)

    def unit_acc_k(w_ref, j, h_in, k, first_layer):
        if first_layer and j == 0:
            h_in[k][...] = jnp.where(
                x_ref[:, k * _T:(k + 1) * _T] >= 0.5,
                1.0, -1.0).astype(_MM_DTYPE)
        for mxu in range(2):
            n = 2 * j + mxu
            pltpu.matmul_push_rhs(
                w_ref[n * _T:(n + 1) * _T, k * _T:(k + 1) * _T],
                staging_register=k % 2, mxu_index=mxu, transpose=True)
            pltpu.matmul_acc_lhs(
                acc_addr=(j % 2) * _HALF,
                lhs=h_in[k][...],
                mxu_index=mxu, load_staged_rhs=k % 2)

    prev = None
    for i, w_ref in enumerate((w1_ref, w2_ref, w3_ref)):
        h_in = bufs[i % 2]
        h_out = bufs[(i + 1) % 2]
        for j in range(2):
            unit_acc_k(w_ref, j, h_in, 0, i == 0)
            unit_acc_k(w_ref, j, h_in, 1, i == 0)
            if prev is not None:
                pop_unit(*prev)
            unit_acc_k(w_ref, j, h_in, 2, i == 0)
            unit_acc_k(w_ref, j, h_in, 3, i == 0)
            prev = (j, i, h_out)

    h_last = bufs[1]
    for kk in range(2):
        pltpu.matmul_push_rhs(
            w4_ref[:, kk * _T:(kk + 1) * _T],
            staging_register=kk, mxu_index=0, transpose=True)
        pltpu.matmul_acc_lhs(
            acc_addr=0, lhs=h_last[kk][...],
            mxu_index=0, load_staged_rhs=kk)
    pop_unit(*prev)
    for kk in range(2):
        k = 2 + kk
        pltpu.matmul_push_rhs(
            w4_ref[:, k * _T:(k + 1) * _T],
            staging_register=kk, mxu_index=1, transpose=True)
        pltpu.matmul_acc_lhs(
            acc_addr=0, lhs=h_last[k][...],
            mxu_index=1, load_staged_rhs=kk)
    tn_w, tn_b, tn_m, tn_v = tn_ref[0], tn_ref[1], tn_ref[2], tn_ref[3]
    c_out = o_ref.shape[1]
    for c in range(chunks):
        y0 = pltpu.matmul_pop(acc_addr=c * (_POP_ROWS // 4),
                              shape=(_POP_ROWS, _T), dtype=jnp.float32,
                              mxu_index=0)
        y1 = pltpu.matmul_pop(acc_addr=c * (_POP_ROWS // 4),
                              shape=(_POP_ROWS, _T), dtype=jnp.float32,
                              mxu_index=1)
        yc = (y0 + y1)[:, :c_out]
        rows = c * _POP_ROWS
        o_ref[rows:rows + _POP_ROWS, :] = (
            (yc - tn_m) * jax.lax.rsqrt(tn_v + TN_EPS) * tn_w + tn_b)


def kernel(x, W1, W2, W3, W4, g1, b1, m1, v1, g2, b2, m2, v2, g3, b3, m3, v3,
           tn_w, tn_b, tn_m, tn_v):
    B, D = x.shape
    H = W1.shape[0]
    C = W4.shape[0]

    w4_padded = jnp.zeros((_T, H), jnp.float32).at[:C].set(W4)
    wb1, wb2, wb3, wb4 = pl.pallas_call(
        _binarize_weights_body,
        out_shape=[
            jax.ShapeDtypeStruct(W1.shape, _MM_DTYPE),
            jax.ShapeDtypeStruct(W2.shape, _MM_DTYPE),
            jax.ShapeDtypeStruct(W3.shape, _MM_DTYPE),
            jax.ShapeDtypeStruct((_T, H), _MM_DTYPE),
        ],
        name="binarize_weights",
    )(W1, W2, W3, w4_padded)

    bn = jnp.stack([g1, b1, m1, v1, g2, b2, m2, v2, g3, b3, m3, v3])
    tn = jnp.stack([tn_w, tn_b, tn_m, tn_v])

    grid = (B // _BM,)
    out = pl.pallas_call(
        _mlp_body,
        grid=grid,
        in_specs=[
            pl.BlockSpec(memory_space=pltpu.SMEM),
            pl.BlockSpec((_BM, D), lambda i: (i, 0)),
            pl.BlockSpec((H, D), lambda i: (0, 0)),
            pl.BlockSpec((H, H), lambda i: (0, 0)),
            pl.BlockSpec((H, H), lambda i: (0, 0)),
            pl.BlockSpec((_T, H), lambda i: (0, 0)),
            pl.BlockSpec((12, H), lambda i: (0, 0)),
        ],
        out_specs=pl.BlockSpec((_BM, C), lambda i: (i, 0)),
        out_shape=jax.ShapeDtypeStruct((B, C), jnp.float32),
        scratch_shapes=[pltpu.VMEM((_BM, _T), _MM_DTYPE)] * 8,
        compiler_params=pltpu.CompilerParams(
            dimension_semantics=("parallel",),
        ),
        name="binary_mlp_fused",
    )(tn, x, wb1, wb2, wb3, wb4, bn)
    return out


# speedup vs baseline: 1.0752x; 1.0752x over previous
import jax
import jax.numpy as jnp
from jax.experimental import pallas as pl
from jax.experimental.pallas import tpu as pltpu

BN_EPS = 1e-5
TN_EPS = 1e-4

_MM_DTYPE = jnp.float8_e4m3fn

_BM = 512
_RU = 256
_POP_ROWS = 64
_T = 256
_Q = 64


def _sign_pm1(x, dtype):
    return jnp.where(x >= 0, 1.0, -1.0).astype(dtype)


def _binarize_weights_body(w1_ref, w2_ref, w3_ref, w4_ref,
                           o1_ref, o2_ref, o3_ref, o4_ref):
    o1_ref[...] = _sign_pm1(w1_ref[...], _MM_DTYPE)
    o2_ref[...] = _sign_pm1(w2_ref[...], _MM_DTYPE)
    o3_ref[...] = _sign_pm1(w3_ref[...], _MM_DTYPE)
    o4_ref[...] = _sign_pm1(w4_ref[...], _MM_DTYPE)


def _mlp_body(tn_ref, x_ref, w1_ref, w2_ref, w3_ref, w4_ref, bn_ref, o_ref,
              *h_refs):
    bufs = (h_refs[:4], h_refs[4:])
    bn_cache = {}
    chunks = _RU // _POP_ROWS

    def bn_params(i, n):
        if (i, n) not in bn_cache:
            cols = n * _T
            g = bn_ref[4 * i + 0, cols:cols + _T]
            b = bn_ref[4 * i + 1, cols:cols + _T]
            m = bn_ref[4 * i + 2, cols:cols + _T]
            v = bn_ref[4 * i + 3, cols:cols + _T]
            bn_cache[(i, n)] = (g * jax.lax.rsqrt(v + BN_EPS), b, m)
        return bn_cache[(i, n)]

    def pop_group(i, j, r):
        q = (2 * j + r) % 4
        h_out = bufs[(i + 1) % 2]
        for mxu in range(2):
            n = 2 * j + mxu
            scale, b, m = bn_params(i, n)
            for c in range(chunks):
                y = pltpu.matmul_pop(
                    acc_addr=q * _Q + c * (_POP_ROWS // 4),
                    shape=(_POP_ROWS, _T), dtype=jnp.float32, mxu_index=mxu)
                t = (y - m) * scale + b
                rows = r * _RU + c * _POP_ROWS
                h_out[n][rows:rows + _POP_ROWS, :] = _sign_pm1(t, _MM_DTYPE)

    pending = []

    def unit(w_ref, i, j, r):
        h_in = bufs[i % 2]
        q = (2 * j + r) % 4
        for k in range(4):
            if i == 0 and j == 0:
                rows = r * _RU
                h_in[k][rows:rows + _RU, :] = jnp.where(
                    x_ref[rows:rows + _RU, k * _T:(k + 1) * _T] >= 0.5,
                    1.0, -1.0).astype(_MM_DTYPE)
            for mxu in range(2):
                n = 2 * j + mxu
                pltpu.matmul_push_rhs(
                    w_ref[n * _T:(n + 1) * _T, k * _T:(k + 1) * _T],
                    staging_register=k % 2, mxu_index=mxu, transpose=True)
                pltpu.matmul_acc_lhs(
                    acc_addr=q * _Q,
                    lhs=h_in[k][r * _RU:(r + 1) * _RU, :],
                    mxu_index=mxu, load_staged_rhs=k % 2)
            if k == 1 and len(pending) > 1:
                pop_group(*pending.pop(0))
        pending.append((i, j, r))

    for i, w_ref in enumerate((w1_ref, w2_ref, w3_ref)):
        for j in range(2):
            for r in range(2):
                unit(w_ref, i, j, r)

    h_last = bufs[1]
    tn_w, tn_b, tn_m, tn_v = tn_ref[0], tn_ref[1], tn_ref[2], tn_ref[3]
    c_out = o_ref.shape[1]

    pop_group(*pending.pop(0))
    for k in range(4):
        pltpu.matmul_push_rhs(
            w4_ref[:, k * _T:(k + 1) * _T],
            staging_register=k % 2, mxu_index=0, transpose=True)
        pltpu.matmul_acc_lhs(
            acc_addr=0, lhs=h_last[k][0:_RU, :],
            mxu_index=0, load_staged_rhs=k % 2)
        if k == 1:
            pop_group(*pending.pop(0))
    for k in range(4):
        pltpu.matmul_push_rhs(
            w4_ref[:, k * _T:(k + 1) * _T],
            staging_register=k % 2, mxu_index=1, transpose=True)
        pltpu.matmul_acc_lhs(
            acc_addr=0, lhs=h_last[k][_RU:2 * _RU, :],
            mxu_index=1, load_staged_rhs=k % 2)

    for mxu in range(2):
        for c in range(chunks):
            y = pltpu.matmul_pop(
                acc_addr=c * (_POP_ROWS // 4),
                shape=(_POP_ROWS, _T), dtype=jnp.float32, mxu_index=mxu)
            yc = y[:, :c_out]
            rows = mxu * _RU + c * _POP_ROWS
            o_ref[rows:rows + _POP_ROWS, :] = (
                (yc - tn_m) * jax.lax.rsqrt(tn_v + TN_EPS) * tn_w + tn_b)


def kernel(x, W1, W2, W3, W4, g1, b1, m1, v1, g2, b2, m2, v2, g3, b3, m3, v3,
           tn_w, tn_b, tn_m, tn_v):
    B, D = x.shape
    H = W1.shape[0]
    C = W4.shape[0]

    w4_padded = jnp.zeros((_T, H), jnp.float32).at[:C].set(W4)
    wb1, wb2, wb3, wb4 = pl.pallas_call(
        _binarize_weights_body,
        out_shape=[
            jax.ShapeDtypeStruct(W1.shape, _MM_DTYPE),
            jax.ShapeDtypeStruct(W2.shape, _MM_DTYPE),
            jax.ShapeDtypeStruct(W3.shape, _MM_DTYPE),
            jax.ShapeDtypeStruct((_T, H), _MM_DTYPE),
        ],
        name="binarize_weights",
    )(W1, W2, W3, w4_padded)

    bn = jnp.stack([g1, b1, m1, v1, g2, b2, m2, v2, g3, b3, m3, v3])
    tn = jnp.stack([tn_w, tn_b, tn_m, tn_v])

    grid = (B // _BM,)
    out = pl.pallas_call(
        _mlp_body,
        grid=grid,
        in_specs=[
            pl.BlockSpec(memory_space=pltpu.SMEM),
            pl.BlockSpec((_BM, D), lambda i: (i, 0)),
            pl.BlockSpec((H, D), lambda i: (0, 0)),
            pl.BlockSpec((H, H), lambda i: (0, 0)),
            pl.BlockSpec((H, H), lambda i: (0, 0)),
            pl.BlockSpec((_T, H), lambda i: (0, 0)),
            pl.BlockSpec((12, H), lambda i: (0, 0)),
        ],
        out_specs=pl.BlockSpec((_BM, C), lambda i: (i, 0)),
        out_shape=jax.ShapeDtypeStruct((B, C), jnp.float32),
        scratch_shapes=[pltpu.VMEM((_BM, _T), _MM_DTYPE)] * 8,
        compiler_params=pltpu.CompilerParams(
            dimension_semantics=("parallel",),
        ),
        name="binary_mlp_fused",
    )(tn, x, wb1, wb2, wb3, wb4, bn)
    return out


# speedup vs baseline: 1.1210x; 1.0426x over previous
import jax
import jax.numpy as jnp
from jax.experimental import pallas as pl
from jax.experimental.pallas import tpu as pltpu

BN_EPS = 1e-5
TN_EPS = 1e-4

_MM_DTYPE = jnp.float8_e4m3fn

_BM = 1024
_RU = 256
_POP_ROWS = 64
_T = 256
_Q = 64


def _sign_pm1(x, dtype):
    return jnp.where(x >= 0, 1.0, -1.0).astype(dtype)


def _binarize_weights_body(w1_ref, w2_ref, w3_ref, w4_ref,
                           o1_ref, o2_ref, o3_ref, o4_ref):
    o1_ref[...] = _sign_pm1(w1_ref[...], _MM_DTYPE)
    o2_ref[...] = _sign_pm1(w2_ref[...], _MM_DTYPE)
    o3_ref[...] = _sign_pm1(w3_ref[...], _MM_DTYPE)
    o4_ref[...] = _sign_pm1(w4_ref[...], _MM_DTYPE)


def _mlp_body(tn_ref, x_ref, w1_ref, w2_ref, w3_ref, w4_ref, bn_ref, o_ref,
              *h_refs):
    bufs = (h_refs[:4], h_refs[4:])
    bn_cache = {}
    chunks = _RU // _POP_ROWS

    def bn_params(i, n):
        if (i, n) not in bn_cache:
            cols = n * _T
            g = bn_ref[4 * i + 0, cols:cols + _T]
            b = bn_ref[4 * i + 1, cols:cols + _T]
            m = bn_ref[4 * i + 2, cols:cols + _T]
            v = bn_ref[4 * i + 3, cols:cols + _T]
            bn_cache[(i, n)] = (g * jax.lax.rsqrt(v + BN_EPS), b, m)
        return bn_cache[(i, n)]

    def pop_group(i, j, r, q):
        h_out = bufs[(i + 1) % 2]
        for mxu in range(2):
            n = 2 * j + mxu
            scale, b, m = bn_params(i, n)
            for c in range(chunks):
                y = pltpu.matmul_pop(
                    acc_addr=q * _Q + c * (_POP_ROWS // 4),
                    shape=(_POP_ROWS, _T), dtype=jnp.float32, mxu_index=mxu)
                t = (y - m) * scale + b
                rows = r * _RU + c * _POP_ROWS
                h_out[n][rows:rows + _POP_ROWS, :] = _sign_pm1(t, _MM_DTYPE)

    pending = []
    uc = [0]

    def unit(w_ref, i, j, r):
        h_in = bufs[i % 2]
        q = uc[0] % 4
        uc[0] += 1
        for k in range(4):
            if i == 0 and j == 0:
                rows = r * _RU
                h_in[k][rows:rows + _RU, :] = jnp.where(
                    x_ref[rows:rows + _RU, k * _T:(k + 1) * _T] >= 0.5,
                    1.0, -1.0).astype(_MM_DTYPE)
            for mxu in range(2):
                n = 2 * j + mxu
                pltpu.matmul_push_rhs(
                    w_ref[n * _T:(n + 1) * _T, k * _T:(k + 1) * _T],
                    staging_register=k % 2, mxu_index=mxu, transpose=True)
                pltpu.matmul_acc_lhs(
                    acc_addr=q * _Q,
                    lhs=h_in[k][r * _RU:(r + 1) * _RU, :],
                    mxu_index=mxu, load_staged_rhs=k % 2)
            if k == 1 and len(pending) > 1:
                pop_group(*pending.pop(0))
        pending.append((i, j, r, q))

    for i, w_ref in enumerate((w1_ref, w2_ref, w3_ref)):
        for j in range(2):
            for r in range(_BM // _RU):
                unit(w_ref, i, j, r)

    h_last = bufs[1]
    tn_w, tn_b, tn_m, tn_v = tn_ref[0], tn_ref[1], tn_ref[2], tn_ref[3]
    c_out = o_ref.shape[1]
    half_rows = _BM // 2

    for k in range(4):
        pltpu.matmul_push_rhs(
            w4_ref[:, k * _T:(k + 1) * _T],
            staging_register=k % 2, mxu_index=0, transpose=True)
        pltpu.matmul_acc_lhs(
            acc_addr=0, lhs=h_last[k][0:half_rows, :],
            mxu_index=0, load_staged_rhs=k % 2)
        if k == 1:
            pop_group(*pending.pop(0))
        if k == 3:
            pop_group(*pending.pop(0))
    for k in range(4):
        pltpu.matmul_push_rhs(
            w4_ref[:, k * _T:(k + 1) * _T],
            staging_register=k % 2, mxu_index=1, transpose=True)
        pltpu.matmul_acc_lhs(
            acc_addr=0, lhs=h_last[k][half_rows:2 * half_rows, :],
            mxu_index=1, load_staged_rhs=k % 2)

    for mxu in range(2):
        for c in range(half_rows // _POP_ROWS):
            y = pltpu.matmul_pop(
                acc_addr=c * (_POP_ROWS // 4),
                shape=(_POP_ROWS, _T), dtype=jnp.float32, mxu_index=mxu)
            yc = y[:, :c_out]
            rows = mxu * half_rows + c * _POP_ROWS
            o_ref[rows:rows + _POP_ROWS, :] = (
                (yc - tn_m) * jax.lax.rsqrt(tn_v + TN_EPS) * tn_w + tn_b)


def kernel(x, W1, W2, W3, W4, g1, b1, m1, v1, g2, b2, m2, v2, g3, b3, m3, v3,
           tn_w, tn_b, tn_m, tn_v):
    B, D = x.shape
    H = W1.shape[0]
    C = W4.shape[0]

    w4_padded = jnp.zeros((_T, H), jnp.float32).at[:C].set(W4)
    wb1, wb2, wb3, wb4 = pl.pallas_call(
        _binarize_weights_body,
        out_shape=[
            jax.ShapeDtypeStruct(W1.shape, _MM_DTYPE),
            jax.ShapeDtypeStruct(W2.shape, _MM_DTYPE),
            jax.ShapeDtypeStruct(W3.shape, _MM_DTYPE),
            jax.ShapeDtypeStruct((_T, H), _MM_DTYPE),
        ],
        name="binarize_weights",
    )(W1, W2, W3, w4_padded)

    bn = jnp.stack([g1, b1, m1, v1, g2, b2, m2, v2, g3, b3, m3, v3])
    tn = jnp.stack([tn_w, tn_b, tn_m, tn_v])

    grid = (B // _BM,)
    out = pl.pallas_call(
        _mlp_body,
        grid=grid,
        in_specs=[
            pl.BlockSpec(memory_space=pltpu.SMEM),
            pl.BlockSpec((_BM, D), lambda i: (i, 0)),
            pl.BlockSpec((H, D), lambda i: (0, 0)),
            pl.BlockSpec((H, H), lambda i: (0, 0)),
            pl.BlockSpec((H, H), lambda i: (0, 0)),
            pl.BlockSpec((_T, H), lambda i: (0, 0)),
            pl.BlockSpec((12, H), lambda i: (0, 0)),
        ],
        out_specs=pl.BlockSpec((_BM, C), lambda i: (i, 0)),
        out_shape=jax.ShapeDtypeStruct((B, C), jnp.float32),
        scratch_shapes=[pltpu.VMEM((_BM, _T), _MM_DTYPE)] * 8,
        compiler_params=pltpu.CompilerParams(
            dimension_semantics=("parallel",),
        ),
        name="binary_mlp_fused",
    )(tn, x, wb1, wb2, wb3, wb4, bn)
    return out


# speedup vs baseline: 1.1676x; 1.0416x over previous
import jax
import jax.numpy as jnp
from jax.experimental import pallas as pl
from jax.experimental.pallas import tpu as pltpu

BN_EPS = 1e-5
TN_EPS = 1e-4

_MM_DTYPE = jnp.float8_e4m3fn

_BM = 1024
_RU = 256
_POP_ROWS = 64
_T = 256
_Q = 64


def _sign_pm1(x, dtype):
    return jnp.where(x >= 0, 1.0, -1.0).astype(dtype)


def _binarize_weights_body(w1_ref, w2_ref, w3_ref, w4_ref,
                           o1_ref, o2_ref, o3_ref, o4_ref):
    o1_ref[...] = _sign_pm1(w1_ref[...], _MM_DTYPE)
    o2_ref[...] = _sign_pm1(w2_ref[...], _MM_DTYPE)
    o3_ref[...] = _sign_pm1(w3_ref[...], _MM_DTYPE)
    o4_ref[...] = _sign_pm1(w4_ref[...], _MM_DTYPE)


def _mlp_body(tn_ref, x_ref, wf1_ref, wf2_ref, wf3_ref, wf4_ref, bn_ref,
              o_ref, w1_ref, w2_ref, w3_ref, w4_ref, *h_refs):
    bufs = (h_refs[:4], h_refs[4:])
    bn_cache = {}
    chunks = _RU // _POP_ROWS

    @pl.when(pl.program_id(0) == 0)
    def _():
        w1_ref[...] = _sign_pm1(wf1_ref[...], _MM_DTYPE)
        w2_ref[...] = _sign_pm1(wf2_ref[...], _MM_DTYPE)
        w3_ref[...] = _sign_pm1(wf3_ref[...], _MM_DTYPE)
        w4_ref[...] = _sign_pm1(wf4_ref[...], _MM_DTYPE)

    def bn_params(i, n):
        if (i, n) not in bn_cache:
            cols = n * _T
            g = bn_ref[4 * i + 0, cols:cols + _T]
            b = bn_ref[4 * i + 1, cols:cols + _T]
            m = bn_ref[4 * i + 2, cols:cols + _T]
            v = bn_ref[4 * i + 3, cols:cols + _T]
            bn_cache[(i, n)] = (g * jax.lax.rsqrt(v + BN_EPS), b, m)
        return bn_cache[(i, n)]

    def pop_group(i, j, r, q):
        h_out = bufs[(i + 1) % 2]
        for mxu in range(2):
            n = 2 * j + mxu
            scale, b, m = bn_params(i, n)
            for c in range(chunks):
                y = pltpu.matmul_pop(
                    acc_addr=q * _Q + c * (_POP_ROWS // 4),
                    shape=(_POP_ROWS, _T), dtype=jnp.float32, mxu_index=mxu)
                t = (y - m) * scale + b
                rows = r * _RU + c * _POP_ROWS
                h_out[n][rows:rows + _POP_ROWS, :] = _sign_pm1(t, _MM_DTYPE)

    pending = []
    uc = [0]

    def unit(w_ref, i, j, r):
        h_in = bufs[i % 2]
        q = uc[0] % 4
        uc[0] += 1
        for k in range(4):
            if i == 0 and j == 0:
                rows = r * _RU
                h_in[k][rows:rows + _RU, :] = jnp.where(
                    x_ref[rows:rows + _RU, k * _T:(k + 1) * _T] >= 0.5,
                    1.0, -1.0).astype(_MM_DTYPE)
            for mxu in range(2):
                n = 2 * j + mxu
                pltpu.matmul_push_rhs(
                    w_ref[n * _T:(n + 1) * _T, k * _T:(k + 1) * _T],
                    staging_register=k % 2, mxu_index=mxu, transpose=True)
                pltpu.matmul_acc_lhs(
                    acc_addr=q * _Q,
                    lhs=h_in[k][r * _RU:(r + 1) * _RU, :],
                    mxu_index=mxu, load_staged_rhs=k % 2)
            if k == 1 and len(pending) > 1:
                pop_group(*pending.pop(0))
        pending.append((i, j, r, q))

    for i, w_ref in enumerate((w1_ref, w2_ref, w3_ref)):
        for j in range(2):
            for r in range(_BM // _RU):
                unit(w_ref, i, j, r)

    h_last = bufs[1]
    tn_w, tn_b, tn_m, tn_v = tn_ref[0], tn_ref[1], tn_ref[2], tn_ref[3]
    c_out = o_ref.shape[1]
    half_rows = _BM // 2

    for k in range(4):
        pltpu.matmul_push_rhs(
            w4_ref[:, k * _T:(k + 1) * _T],
            staging_register=k % 2, mxu_index=0, transpose=True)
        pltpu.matmul_acc_lhs(
            acc_addr=0, lhs=h_last[k][0:half_rows, :],
            mxu_index=0, load_staged_rhs=k % 2)
        if k == 1:
            pop_group(*pending.pop(0))
        if k == 3:
            pop_group(*pending.pop(0))
    for k in range(4):
        pltpu.matmul_push_rhs(
            w4_ref[:, k * _T:(k + 1) * _T],
            staging_register=k % 2, mxu_index=1, transpose=True)
        pltpu.matmul_acc_lhs(
            acc_addr=0, lhs=h_last[k][half_rows:2 * half_rows, :],
            mxu_index=1, load_staged_rhs=k % 2)

    for mxu in range(2):
        for c in range(half_rows // _POP_ROWS):
            y = pltpu.matmul_pop(
                acc_addr=c * (_POP_ROWS // 4),
                shape=(_POP_ROWS, _T), dtype=jnp.float32, mxu_index=mxu)
            yc = y[:, :c_out]
            rows = mxu * half_rows + c * _POP_ROWS
            o_ref[rows:rows + _POP_ROWS, :] = (
                (yc - tn_m) * jax.lax.rsqrt(tn_v + TN_EPS) * tn_w + tn_b)


def kernel(x, W1, W2, W3, W4, g1, b1, m1, v1, g2, b2, m2, v2, g3, b3, m3, v3,
           tn_w, tn_b, tn_m, tn_v):
    B, D = x.shape
    H = W1.shape[0]
    C = W4.shape[0]

    w4_padded = jnp.zeros((_T, H), jnp.float32).at[:C].set(W4)
    bn = jnp.stack([g1, b1, m1, v1, g2, b2, m2, v2, g3, b3, m3, v3])
    tn = jnp.stack([tn_w, tn_b, tn_m, tn_v])

    grid = (B // _BM,)
    out = pl.pallas_call(
        _mlp_body,
        grid=grid,
        in_specs=[
            pl.BlockSpec(memory_space=pltpu.SMEM),
            pl.BlockSpec((_BM, D), lambda i: (i, 0)),
            pl.BlockSpec((H, D), lambda i: (0, 0)),
            pl.BlockSpec((H, H), lambda i: (0, 0)),
            pl.BlockSpec((H, H), lambda i: (0, 0)),
            pl.BlockSpec((_T, H), lambda i: (0, 0)),
            pl.BlockSpec((12, H), lambda i: (0, 0)),
        ],
        out_specs=pl.BlockSpec((_BM, C), lambda i: (i, 0)),
        out_shape=jax.ShapeDtypeStruct((B, C), jnp.float32),
        scratch_shapes=(
            [pltpu.VMEM((H, D), _MM_DTYPE)] * 3
            + [pltpu.VMEM((_T, H), _MM_DTYPE)]
            + [pltpu.VMEM((_BM, _T), _MM_DTYPE)] * 8
        ),
        compiler_params=pltpu.CompilerParams(
            dimension_semantics=("arbitrary",),
        ),
        name="binary_mlp_fused",
    )(tn, x, W1, W2, W3, w4_padded, bn)
    return out


# speedup vs baseline: 1.2110x; 1.0371x over previous
import jax
import jax.numpy as jnp
from jax.experimental import pallas as pl
from jax.experimental.pallas import tpu as pltpu

BN_EPS = 1e-5
TN_EPS = 1e-4

_MM_DTYPE = jnp.float8_e4m3fn

_BM = 1024
_RU = 256
_POP_ROWS = 64
_T = 256
_Q = 64


def _sign_pm1(x, dtype):
    return jnp.where(x >= 0, 1.0, -1.0).astype(dtype)


def _mlp_body(tnw_ref, tnb_ref, tnm_ref, tnv_ref, x_ref,
              wf1_ref, wf2_ref, wf3_ref, wf4_ref, *rest):
    bn_refs = rest[:12]
    o_ref = rest[12]
    w1_ref, w2_ref, w3_ref, w4_ref = rest[13:17]
    h_refs = rest[17:]
    bufs = (h_refs[:4], h_refs[4:])
    bn_cache = {}
    chunks = _RU // _POP_ROWS

    @pl.when(pl.program_id(0) == 0)
    def _():
        w1_ref[...] = _sign_pm1(wf1_ref[...], _MM_DTYPE)
        w2_ref[...] = _sign_pm1(wf2_ref[...], _MM_DTYPE)
        w3_ref[...] = _sign_pm1(wf3_ref[...], _MM_DTYPE)
        w4_ref[...] = _sign_pm1(wf4_ref[...], _MM_DTYPE)

    def bn_params(i, n):
        if (i, n) not in bn_cache:
            cols = n * _T
            g = bn_refs[4 * i + 0][0, cols:cols + _T]
            b = bn_refs[4 * i + 1][0, cols:cols + _T]
            m = bn_refs[4 * i + 2][0, cols:cols + _T]
            v = bn_refs[4 * i + 3][0, cols:cols + _T]
            bn_cache[(i, n)] = (g * jax.lax.rsqrt(v + BN_EPS), b, m)
        return bn_cache[(i, n)]

    def pop_group(i, j, r, q):
        h_out = bufs[(i + 1) % 2]
        for mxu in range(2):
            n = 2 * j + mxu
            scale, b, m = bn_params(i, n)
            for c in range(chunks):
                y = pltpu.matmul_pop(
                    acc_addr=q * _Q + c * (_POP_ROWS // 4),
                    shape=(_POP_ROWS, _T), dtype=jnp.float32, mxu_index=mxu)
                t = (y - m) * scale + b
                rows = r * _RU + c * _POP_ROWS
                h_out[n][rows:rows + _POP_ROWS, :] = _sign_pm1(t, _MM_DTYPE)

    pending = []
    uc = [0]

    def unit(w_ref, i, j, r):
        h_in = bufs[i % 2]
        q = uc[0] % 4
        uc[0] += 1
        for k in range(4):
            if i == 0 and j == 0:
                rows = r * _RU
                h_in[k][rows:rows + _RU, :] = jnp.where(
                    x_ref[rows:rows + _RU, k * _T:(k + 1) * _T] >= 0.5,
                    1.0, -1.0).astype(_MM_DTYPE)
            for mxu in range(2):
                n = 2 * j + mxu
                pltpu.matmul_push_rhs(
                    w_ref[n * _T:(n + 1) * _T, k * _T:(k + 1) * _T],
                    staging_register=k % 2, mxu_index=mxu, transpose=True)
                pltpu.matmul_acc_lhs(
                    acc_addr=q * _Q,
                    lhs=h_in[k][r * _RU:(r + 1) * _RU, :],
                    mxu_index=mxu, load_staged_rhs=k % 2)
            if k == 1 and len(pending) > 1:
                pop_group(*pending.pop(0))
        pending.append((i, j, r, q))

    for i, w_ref in enumerate((w1_ref, w2_ref, w3_ref)):
        for j in range(2):
            for r in range(_BM // _RU):
                unit(w_ref, i, j, r)

    h_last = bufs[1]
    tn_w, tn_b = tnw_ref[0], tnb_ref[0]
    tn_m, tn_v = tnm_ref[0], tnv_ref[0]
    c_out = o_ref.shape[1]
    half_rows = _BM // 2

    for k in range(4):
        pltpu.matmul_push_rhs(
            w4_ref[:, k * _T:(k + 1) * _T],
            staging_register=k % 2, mxu_index=0, transpose=True)
        pltpu.matmul_acc_lhs(
            acc_addr=0, lhs=h_last[k][0:half_rows, :],
            mxu_index=0, load_staged_rhs=k % 2)
        if k == 1:
            pop_group(*pending.pop(0))
        if k == 3:
            pop_group(*pending.pop(0))
    for k in range(4):
        pltpu.matmul_push_rhs(
            w4_ref[:, k * _T:(k + 1) * _T],
            staging_register=k % 2, mxu_index=1, transpose=True)
        pltpu.matmul_acc_lhs(
            acc_addr=0, lhs=h_last[k][half_rows:2 * half_rows, :],
            mxu_index=1, load_staged_rhs=k % 2)

    for mxu in range(2):
        for c in range(half_rows // _POP_ROWS):
            y = pltpu.matmul_pop(
                acc_addr=c * (_POP_ROWS // 4),
                shape=(_POP_ROWS, _T), dtype=jnp.float32, mxu_index=mxu)
            yc = y[:, :c_out]
            rows = mxu * half_rows + c * _POP_ROWS
            o_ref[rows:rows + _POP_ROWS, :] = (
                (yc - tn_m) * jax.lax.rsqrt(tn_v + TN_EPS) * tn_w + tn_b)


def kernel(x, W1, W2, W3, W4, g1, b1, m1, v1, g2, b2, m2, v2, g3, b3, m3, v3,
           tn_w, tn_b, tn_m, tn_v):
    B, D = x.shape
    H = W1.shape[0]
    C = W4.shape[0]

    w4_padded = jnp.zeros((_T, H), jnp.float32).at[:C].set(W4)
    bn_rows = [jnp.reshape(a, (1, H)) for a in
               (g1, b1, m1, v1, g2, b2, m2, v2, g3, b3, m3, v3)]
    tn_scalars = [jnp.reshape(a, (1,)) for a in (tn_w, tn_b, tn_m, tn_v)]

    grid = (B // _BM,)
    out = pl.pallas_call(
        _mlp_body,
        grid=grid,
        in_specs=(
            [pl.BlockSpec(memory_space=pltpu.SMEM)] * 4
            + [
                pl.BlockSpec((_BM, D), lambda i: (i, 0)),
                pl.BlockSpec((H, D), lambda i: (0, 0)),
                pl.BlockSpec((H, H), lambda i: (0, 0)),
                pl.BlockSpec((H, H), lambda i: (0, 0)),
                pl.BlockSpec((_T, H), lambda i: (0, 0)),
            ]
            + [pl.BlockSpec((1, H), lambda i: (0, 0))] * 12
        ),
        out_specs=pl.BlockSpec((_BM, C), lambda i: (i, 0)),
        out_shape=jax.ShapeDtypeStruct((B, C), jnp.float32),
        scratch_shapes=(
            [pltpu.VMEM((H, D), _MM_DTYPE)] * 3
            + [pltpu.VMEM((_T, H), _MM_DTYPE)]
            + [pltpu.VMEM((_BM, _T), _MM_DTYPE)] * 8
        ),
        compiler_params=pltpu.CompilerParams(
            dimension_semantics=("arbitrary",),
        ),
        name="binary_mlp_fused",
    )(*tn_scalars, x, W1, W2, W3, w4_padded, *bn_rows)
    return out


# speedup vs baseline: 1.2343x; 1.0193x over previous
import jax
import jax.numpy as jnp
from jax.experimental import pallas as pl
from jax.experimental.pallas import tpu as pltpu

BN_EPS = 1e-5
TN_EPS = 1e-4

_MM_DTYPE = jnp.float8_e4m3fn

_BM = 2048
_RU = 256
_POP_ROWS = 64
_T = 256
_Q = 64


def _sign_pm1(x, dtype):
    return jnp.where(x >= 0, 1.0, -1.0).astype(dtype)


def _mlp_body(tnw_ref, tnb_ref, tnm_ref, tnv_ref, x_ref,
              wf1_ref, wf2_ref, wf3_ref, wf4_ref, *rest):
    bn_refs = rest[:12]
    o_ref = rest[12]
    w1_ref, w2_ref, w3_ref, w4_ref = rest[13:17]
    h_refs = rest[17:]
    bufs = (h_refs[:4], h_refs[4:])
    bn_cache = {}
    chunks = _RU // _POP_ROWS

    @pl.when(pl.program_id(0) == 0)
    def _():
        w1_ref[...] = _sign_pm1(wf1_ref[...], _MM_DTYPE)
        w2_ref[...] = _sign_pm1(wf2_ref[...], _MM_DTYPE)
        w3_ref[...] = _sign_pm1(wf3_ref[...], _MM_DTYPE)
        w4_ref[...] = _sign_pm1(wf4_ref[...], _MM_DTYPE)

    def bn_params(i, n):
        if (i, n) not in bn_cache:
            cols = n * _T
            g = bn_refs[4 * i + 0][0, cols:cols + _T]
            b = bn_refs[4 * i + 1][0, cols:cols + _T]
            m = bn_refs[4 * i + 2][0, cols:cols + _T]
            v = bn_refs[4 * i + 3][0, cols:cols + _T]
            bn_cache[(i, n)] = (g * jax.lax.rsqrt(v + BN_EPS), b, m)
        return bn_cache[(i, n)]

    def pop_group(i, j, r, q):
        h_out = bufs[(i + 1) % 2]
        for mxu in range(2):
            n = 2 * j + mxu
            scale, b, m = bn_params(i, n)
            for c in range(chunks):
                y = pltpu.matmul_pop(
                    acc_addr=q * _Q + c * (_POP_ROWS // 4),
                    shape=(_POP_ROWS, _T), dtype=jnp.float32, mxu_index=mxu)
                t = (y - m) * scale + b
                rows = r * _RU + c * _POP_ROWS
                h_out[n][rows:rows + _POP_ROWS, :] = _sign_pm1(t, _MM_DTYPE)

    pending = []
    uc = [0]

    def unit(w_ref, i, j, r):
        h_in = bufs[i % 2]
        q = uc[0] % 4
        uc[0] += 1
        for k in range(4):
            if i == 0 and j == 0:
                rows = r * _RU
                h_in[k][rows:rows + _RU, :] = jnp.where(
                    x_ref[rows:rows + _RU, k * _T:(k + 1) * _T] >= 0.5,
                    1.0, -1.0).astype(_MM_DTYPE)
            for mxu in range(2):
                n = 2 * j + mxu
                pltpu.matmul_push_rhs(
                    w_ref[n * _T:(n + 1) * _T, k * _T:(k + 1) * _T],
                    staging_register=k % 2, mxu_index=mxu, transpose=True)
                pltpu.matmul_acc_lhs(
                    acc_addr=q * _Q,
                    lhs=h_in[k][r * _RU:(r + 1) * _RU, :],
                    mxu_index=mxu, load_staged_rhs=k % 2)
            if k == 1 and len(pending) > 1:
                pop_group(*pending.pop(0))
        pending.append((i, j, r, q))

    for i, w_ref in enumerate((w1_ref, w2_ref, w3_ref)):
        for j in range(2):
            for r in range(_BM // _RU):
                unit(w_ref, i, j, r)

    h_last = bufs[1]
    tn_w, tn_b = tnw_ref[0], tnb_ref[0]
    tn_m, tn_v = tnm_ref[0], tnv_ref[0]
    c_out = o_ref.shape[1]
    half_rows = _BM // 2

    for rnd in range(2):
        base = rnd * 128
        for k in range(4):
            for mxu in range(2):
                pltpu.matmul_push_rhs(
                    w4_ref[:, k * _T:(k + 1) * _T],
                    staging_register=k % 2, mxu_index=mxu, transpose=True)
                rows = mxu * half_rows + rnd * 512
                pltpu.matmul_acc_lhs(
                    acc_addr=base, lhs=h_last[k][rows:rows + 512, :],
                    mxu_index=mxu, load_staged_rhs=k % 2)
            if rnd == 0 and k == 1:
                pop_group(*pending.pop(0))
            if rnd == 0 and k == 3:
                pop_group(*pending.pop(0))

    for rnd in range(2):
        base = rnd * 128
        for mxu in range(2):
            for c in range(512 // _POP_ROWS):
                y = pltpu.matmul_pop(
                    acc_addr=base + c * (_POP_ROWS // 4),
                    shape=(_POP_ROWS, _T), dtype=jnp.float32, mxu_index=mxu)
                yc = y[:, :c_out]
                rows = mxu * half_rows + rnd * 512 + c * _POP_ROWS
                o_ref[rows:rows + _POP_ROWS, :] = (
                    (yc - tn_m) * jax.lax.rsqrt(tn_v + TN_EPS) * tn_w + tn_b)


def kernel(x, W1, W2, W3, W4, g1, b1, m1, v1, g2, b2, m2, v2, g3, b3, m3, v3,
           tn_w, tn_b, tn_m, tn_v):
    B, D = x.shape
    H = W1.shape[0]
    C = W4.shape[0]

    w4_padded = jnp.zeros((_T, H), jnp.float32).at[:C].set(W4)
    bn_rows = [jnp.reshape(a, (1, H)) for a in
               (g1, b1, m1, v1, g2, b2, m2, v2, g3, b3, m3, v3)]
    tn_scalars = [jnp.reshape(a, (1,)) for a in (tn_w, tn_b, tn_m, tn_v)]

    grid = (B // _BM,)
    out = pl.pallas_call(
        _mlp_body,
        grid=grid,
        in_specs=(
            [pl.BlockSpec(memory_space=pltpu.SMEM)] * 4
            + [
                pl.BlockSpec((_BM, D), lambda i: (i, 0)),
                pl.BlockSpec((H, D), lambda i: (0, 0)),
                pl.BlockSpec((H, H), lambda i: (0, 0)),
                pl.BlockSpec((H, H), lambda i: (0, 0)),
                pl.BlockSpec((_T, H), lambda i: (0, 0)),
            ]
            + [pl.BlockSpec((1, H), lambda i: (0, 0))] * 12
        ),
        out_specs=pl.BlockSpec((_BM, C), lambda i: (i, 0)),
        out_shape=jax.ShapeDtypeStruct((B, C), jnp.float32),
        scratch_shapes=(
            [pltpu.VMEM((H, D), _MM_DTYPE)] * 3
            + [pltpu.VMEM((_T, H), _MM_DTYPE)]
            + [pltpu.VMEM((_BM, _T), _MM_DTYPE)] * 8
        ),
        compiler_params=pltpu.CompilerParams(
            dimension_semantics=("arbitrary",),
        ),
        name="binary_mlp_fused",
    )(*tn_scalars, x, W1, W2, W3, w4_padded, *bn_rows)
    return out
